# Initial kernel scaffold; baseline (speedup 1.0000x reference)
import functools

import jax
import jax.numpy as jnp
from jax import lax
from jax.experimental import pallas as pl
from jax.experimental.pallas import tpu as pltpu
from jax.experimental.pallas import tpu_sc as plsc

B, N, D = 1024, 200, 128
BN = B * N

_info = plsc.get_sparse_core_info()
NC, NS = _info.num_cores, _info.num_subcores
NW = NC * NS

CHUNK = 128
ROWS_W = BN // NW
NCHUNK = ROWS_W // CHUNK
SB = B // NW

_mesh = plsc.VectorSubcoreMesh(core_axis_name="c", subcore_axis_name="s")


@functools.partial(
    pl.kernel,
    mesh=_mesh,
    out_type=(
        jax.ShapeDtypeStruct((BN, D), jnp.float32),
        jax.ShapeDtypeStruct((B, D), jnp.float32),
        jax.ShapeDtypeStruct((B, D), jnp.float32),
    ),
    scratch_types=[
        pltpu.VMEM((NCHUNK, CHUNK), jnp.int32),
        pltpu.VMEM((CHUNK, D), jnp.float32),
        pltpu.VMEM((SB,), jnp.int32),
        pltpu.VMEM((SB, D), jnp.float32),
        pltpu.SemaphoreType.DMA,
    ],
)
def _sc_gather(p_hbm, s_hbm, k_hbm, emb_idx_hbm, emb_k_hbm,
               mem_out, qs_out, qk_out,
               idx_v, rows_v, sidx_v, srows_v, sem):
    wid = lax.axis_index("s") * NC + lax.axis_index("c")
    base = wid * ROWS_W

    pltpu.sync_copy(p_hbm.at[pl.ds(wid * NCHUNK, NCHUNK)], idx_v)

    def chunk(c, carry):
        pltpu.async_copy(emb_idx_hbm.at[idx_v.at[c]], rows_v, sem).wait()
        pltpu.sync_copy(rows_v, mem_out.at[pl.ds(base + c * CHUNK, CHUNK)])
        return carry

    lax.fori_loop(0, NCHUNK, chunk, 0)

    pltpu.sync_copy(s_hbm.at[pl.ds(wid * SB, SB)], sidx_v)
    pltpu.async_copy(emb_idx_hbm.at[sidx_v], srows_v, sem).wait()
    pltpu.sync_copy(srows_v, qs_out.at[pl.ds(wid * SB, SB)])

    pltpu.sync_copy(k_hbm.at[pl.ds(wid * SB, SB)], sidx_v)
    pltpu.async_copy(emb_k_hbm.at[sidx_v], srows_v, sem).wait()
    pltpu.sync_copy(srows_v, qk_out.at[pl.ds(wid * SB, SB)])


def kernel(p, s, k, emb_idx, emb_k):
    p2d = p.astype(jnp.int32).reshape(BN // CHUNK, CHUNK)
    mem, q_s, q_k = _sc_gather(
        p2d, s.astype(jnp.int32), k.astype(jnp.int32),
        emb_idx.astype(jnp.float32), emb_k.astype(jnp.float32))
    return mem.reshape(B, N, D), q_s, q_k


# speedup vs baseline: 3.0358x; 3.0358x over previous
import functools

import jax
import jax.numpy as jnp
from jax import lax
from jax.experimental import pallas as pl
from jax.experimental.pallas import tpu as pltpu
from jax.experimental.pallas import tpu_sc as plsc

B, N, D = 1024, 200, 128
BN = B * N

_info = plsc.get_sparse_core_info()
NC, NS = _info.num_cores, _info.num_subcores
NW = NC * NS

CHUNK = 128
ROWS_W = BN // NW
NCHUNK = ROWS_W // CHUNK
SB = B // NW

_mesh = plsc.VectorSubcoreMesh(core_axis_name="c", subcore_axis_name="s")


@functools.partial(
    pl.kernel,
    mesh=_mesh,
    out_type=(
        jax.ShapeDtypeStruct((BN, D), jnp.float32),
        jax.ShapeDtypeStruct((B, D), jnp.float32),
        jax.ShapeDtypeStruct((B, D), jnp.float32),
    ),
    scratch_types=[
        pltpu.VMEM((ROWS_W,), jnp.int32),
        pltpu.VMEM((CHUNK, D), jnp.float32),
        pltpu.VMEM((SB,), jnp.int32),
        pltpu.VMEM((SB, D), jnp.float32),
        pltpu.SemaphoreType.DMA,
    ],
)
def _sc_gather(p_hbm, s_hbm, k_hbm, emb_idx_hbm, emb_k_hbm,
               mem_out, qs_out, qk_out,
               idx_v, rows_v, sidx_v, srows_v, sem):
    wid = lax.axis_index("s") * NC + lax.axis_index("c")
    base = wid * ROWS_W

    pltpu.sync_copy(p_hbm.at[pl.ds(base, ROWS_W)], idx_v)

    def chunk(c, carry):
        pltpu.async_copy(
            emb_idx_hbm.at[idx_v.at[pl.ds(c * CHUNK, CHUNK)]], rows_v, sem
        ).wait()
        pltpu.sync_copy(rows_v, mem_out.at[pl.ds(base + c * CHUNK, CHUNK)])
        return carry

    lax.fori_loop(0, NCHUNK, chunk, 0)

    pltpu.sync_copy(s_hbm.at[pl.ds(wid * SB, SB)], sidx_v)
    pltpu.async_copy(emb_idx_hbm.at[sidx_v], srows_v, sem).wait()
    pltpu.sync_copy(srows_v, qs_out.at[pl.ds(wid * SB, SB)])

    pltpu.sync_copy(k_hbm.at[pl.ds(wid * SB, SB)], sidx_v)
    pltpu.async_copy(emb_k_hbm.at[sidx_v], srows_v, sem).wait()
    pltpu.sync_copy(srows_v, qk_out.at[pl.ds(wid * SB, SB)])


def kernel(p, s, k, emb_idx, emb_k):
    p1d = p.astype(jnp.int32).reshape(BN)
    mem, q_s, q_k = _sc_gather(
        p1d, s.astype(jnp.int32), k.astype(jnp.int32),
        emb_idx.astype(jnp.float32), emb_k.astype(jnp.float32))
    return mem.reshape(B, N, D), q_s, q_k


# speedup vs baseline: 3.0456x; 1.0032x over previous
import functools

import jax
import jax.numpy as jnp
from jax import lax
from jax.experimental import pallas as pl
from jax.experimental.pallas import tpu as pltpu
from jax.experimental.pallas import tpu_sc as plsc

B, N, D = 1024, 200, 128
BN = B * N

_info = plsc.get_sparse_core_info()
NC, NS = _info.num_cores, _info.num_subcores
NW = NC * NS

CHUNK = 128
ROWS_W = BN // NW
NCHUNK = ROWS_W // CHUNK
NBUF = 5
NOUTER = NCHUNK // NBUF
SB = B // NW

_mesh = plsc.VectorSubcoreMesh(core_axis_name="c", subcore_axis_name="s")


@functools.partial(
    pl.kernel,
    mesh=_mesh,
    out_type=(
        jax.ShapeDtypeStruct((BN, D), jnp.float32),
        jax.ShapeDtypeStruct((B, D), jnp.float32),
        jax.ShapeDtypeStruct((B, D), jnp.float32),
    ),
    scratch_types=[
        pltpu.VMEM((ROWS_W,), jnp.int32),
        pltpu.VMEM((NBUF, CHUNK, D), jnp.float32),
        pltpu.VMEM((SB,), jnp.int32),
        pltpu.VMEM((SB,), jnp.int32),
        pltpu.VMEM((SB, D), jnp.float32),
        pltpu.VMEM((SB, D), jnp.float32),
        pltpu.SemaphoreType.DMA((NBUF,)),
        pltpu.SemaphoreType.DMA((NBUF,)),
        pltpu.SemaphoreType.DMA,
    ],
)
def _sc_gather(p_hbm, s_hbm, k_hbm, emb_idx_hbm, emb_k_hbm,
               mem_out, qs_out, qk_out,
               idx_v, bufs, sidx_v, kidx_v, srows_v, krows_v,
               gsem, ssem, qsem):
    wid = lax.axis_index("s") * NC + lax.axis_index("c")
    base = wid * ROWS_W

    pltpu.sync_copy(p_hbm.at[pl.ds(base, ROWS_W)], idx_v)

    pltpu.sync_copy(s_hbm.at[pl.ds(wid * SB, SB)], sidx_v)
    pltpu.sync_copy(k_hbm.at[pl.ds(wid * SB, SB)], kidx_v)
    qs_gather = pltpu.async_copy(emb_idx_hbm.at[sidx_v], srows_v, qsem)
    qk_gather = pltpu.async_copy(emb_k_hbm.at[kidx_v], krows_v, qsem)

    def outer(g, carry):
        c0 = g * NBUF
        gathers = []
        for b in range(NBUF):
            @pl.when(g > 0)
            def _drain(b=b):
                pltpu.make_async_copy(
                    bufs.at[b], mem_out.at[pl.ds(base, CHUNK)], ssem.at[b]
                ).wait()

            gathers.append(pltpu.async_copy(
                emb_idx_hbm.at[idx_v.at[pl.ds((c0 + b) * CHUNK, CHUNK)]],
                bufs.at[b], gsem.at[b]))
        for b in range(NBUF):
            gathers[b].wait()
            pltpu.async_copy(
                bufs.at[b],
                mem_out.at[pl.ds(base + (c0 + b) * CHUNK, CHUNK)],
                ssem.at[b])
        return carry

    lax.fori_loop(0, NOUTER, outer, 0)

    for b in range(NBUF):
        pltpu.make_async_copy(
            bufs.at[b], mem_out.at[pl.ds(base, CHUNK)], ssem.at[b]).wait()

    qs_gather.wait()
    qk_gather.wait()
    pltpu.sync_copy(srows_v, qs_out.at[pl.ds(wid * SB, SB)])
    pltpu.sync_copy(krows_v, qk_out.at[pl.ds(wid * SB, SB)])


def kernel(p, s, k, emb_idx, emb_k):
    p1d = p.astype(jnp.int32).reshape(BN)
    mem, q_s, q_k = _sc_gather(
        p1d, s.astype(jnp.int32), k.astype(jnp.int32),
        emb_idx.astype(jnp.float32), emb_k.astype(jnp.float32))
    return mem.reshape(B, N, D), q_s, q_k


# speedup vs baseline: 11.5221x; 3.7832x over previous
import functools

import jax
import jax.numpy as jnp
from jax import lax
from jax.experimental import pallas as pl
from jax.experimental.pallas import tpu as pltpu
from jax.experimental.pallas import tpu_sc as plsc

B, N, D = 1024, 200, 128
BN = B * N

_info = plsc.get_sparse_core_info()
NC, NS = _info.num_cores, _info.num_subcores
NW = NC * NS

CHUNK = 128
ROWS_W = BN // NW
NCHUNK = ROWS_W // CHUNK
NBUF = 5
NOUTER = NCHUNK // NBUF
SB = B // NW

_mesh = plsc.VectorSubcoreMesh(core_axis_name="c", subcore_axis_name="s")


@functools.partial(
    pl.kernel,
    mesh=_mesh,
    out_type=(
        jax.ShapeDtypeStruct((BN, D), jnp.float32),
        jax.ShapeDtypeStruct((B, D), jnp.float32),
        jax.ShapeDtypeStruct((B, D), jnp.float32),
    ),
    scratch_types=[
        pltpu.VMEM((ROWS_W,), jnp.int32),
        pltpu.VMEM((NBUF, CHUNK, D), jnp.float32),
        pltpu.VMEM((SB,), jnp.int32),
        pltpu.VMEM((SB,), jnp.int32),
        pltpu.VMEM((SB, D), jnp.float32),
        pltpu.VMEM((SB, D), jnp.float32),
        pltpu.SemaphoreType.DMA((NBUF,)),
        pltpu.SemaphoreType.DMA((NBUF,)),
        pltpu.SemaphoreType.DMA,
        pltpu.VMEM_SHARED((N, D), jnp.float32),
    ],
)
def _sc_gather(p_hbm, s_hbm, k_hbm, emb_idx_hbm, emb_k_hbm,
               mem_out, qs_out, qk_out,
               idx_v, bufs, sidx_v, kidx_v, srows_v, krows_v,
               gsem, ssem, qsem, table_sh):
    wid = lax.axis_index("s") * NC + lax.axis_index("c")
    base = wid * ROWS_W

    @pl.when(lax.axis_index("s") == 0)
    def _stage_table():
        pltpu.sync_copy(emb_idx_hbm, table_sh)

    pltpu.sync_copy(p_hbm.at[pl.ds(base, ROWS_W)], idx_v)
    plsc.subcore_barrier()

    pltpu.sync_copy(s_hbm.at[pl.ds(wid * SB, SB)], sidx_v)
    pltpu.sync_copy(k_hbm.at[pl.ds(wid * SB, SB)], kidx_v)
    qs_gather = pltpu.async_copy(emb_idx_hbm.at[sidx_v], srows_v, qsem)
    qk_gather = pltpu.async_copy(emb_k_hbm.at[kidx_v], krows_v, qsem)

    def outer(g, carry):
        c0 = g * NBUF
        gathers = []
        for b in range(NBUF):
            @pl.when(g > 0)
            def _drain(b=b):
                pltpu.make_async_copy(
                    bufs.at[b], mem_out.at[pl.ds(base, CHUNK)], ssem.at[b]
                ).wait()

            gathers.append(pltpu.async_copy(
                table_sh.at[idx_v.at[pl.ds((c0 + b) * CHUNK, CHUNK)]],
                bufs.at[b], gsem.at[b]))
        for b in range(NBUF):
            gathers[b].wait()
            pltpu.async_copy(
                bufs.at[b],
                mem_out.at[pl.ds(base + (c0 + b) * CHUNK, CHUNK)],
                ssem.at[b])
        return carry

    lax.fori_loop(0, NOUTER, outer, 0)

    for b in range(NBUF):
        pltpu.make_async_copy(
            bufs.at[b], mem_out.at[pl.ds(base, CHUNK)], ssem.at[b]).wait()

    qs_gather.wait()
    qk_gather.wait()
    pltpu.sync_copy(srows_v, qs_out.at[pl.ds(wid * SB, SB)])
    pltpu.sync_copy(krows_v, qk_out.at[pl.ds(wid * SB, SB)])


def kernel(p, s, k, emb_idx, emb_k):
    p1d = p.astype(jnp.int32).reshape(BN)
    mem, q_s, q_k = _sc_gather(
        p1d, s.astype(jnp.int32), k.astype(jnp.int32),
        emb_idx.astype(jnp.float32), emb_k.astype(jnp.float32))
    return mem.reshape(B, N, D), q_s, q_k
